# baseline (speedup 1.0000x reference)
import jax
import jax.numpy as jnp
from jax import lax
from jax.experimental import pallas as pl
from jax.experimental.pallas import tpu as pltpu
from jax.experimental.pallas import tpu_sc as plsc

N = 10000
NP = 10240
E = 320000
D = 128

NC = 2
NS = 16
L = 16

EB = 80
CH = 2000
CB = CH // EB
SCH = 2000
TILE_E = E // (NC * NS)
NCHUNK_A = TILE_E // SCH
NCHUNK_B = TILE_E // CH
SEG = NP // NS

BLK_E = 2000
BLK_N = 1000


def _elu(x):
    return jnp.where(x > 0, x, jnp.exp(jnp.minimum(x, 0.0)) - 1.0)


def _kw_body(off_ref, w_ref, kW0_ref, kb0_ref, kW1_ref, kb1_ref, out_ref):
    off = off_ref[...]
    w0 = kW0_ref[...]
    h = off[:, 0:1] * w0[0:1, :] + off[:, 1:2] * w0[1:2, :] + kb0_ref[...]
    h = _elu(h)
    kw = jnp.dot(h, kW1_ref[...], preferred_element_type=jnp.float32) + kb1_ref[...]
    out_ref[...] = kw * w_ref[...]


def _kw_pallas(offsets, w, kW0, kb0, kW1, kb1):
    return pl.pallas_call(
        _kw_body,
        grid=(E // BLK_E,),
        in_specs=[
            pl.BlockSpec((BLK_E, 2), lambda i: (i, 0)),
            pl.BlockSpec((BLK_E, 1), lambda i: (i, 0)),
            pl.BlockSpec((2, D), lambda i: (0, 0)),
            pl.BlockSpec((1, D), lambda i: (0, 0)),
            pl.BlockSpec((D, D), lambda i: (0, 0)),
            pl.BlockSpec((1, D), lambda i: (0, 0)),
        ],
        out_specs=pl.BlockSpec((BLK_E, D), lambda i: (i, 0)),
        out_shape=jax.ShapeDtypeStruct((E, D), jnp.float32),
    )(offsets, w.reshape(E, 1), kW0, kb0.reshape(1, D), kW1, kb1.reshape(1, D))


def _g16(x, idx):
    return lax.gather(
        x, idx[:, None],
        lax.GatherDimensionNumbers(
            offset_dims=(), collapsed_slice_dims=(0,), start_index_map=(0,)),
        (1,), mode=lax.GatherScatterMode.PROMISE_IN_BOUNDS)


def _seg_rmw(tab_ref, d16, v16, is_max):
    iota = lax.iota(jnp.int32, L)
    sd, sv = plsc.sort_key_val(d16, v16)
    for k in (1, 2, 4, 8):
        idx = jnp.maximum(iota - k, 0)
        sh_v = _g16(sv, idx)
        sh_d = _g16(sd, idx)
        m = (sh_d == sd) & (iota >= k)
        if is_max:
            sv = jnp.where(m, jnp.maximum(sv, sh_v), sv)
        else:
            sv = jnp.where(m, sv + sh_v, sv)
    nxt = _g16(sd, jnp.minimum(iota + 1, L - 1))
    lastm = (nxt != sd) | (iota == L - 1)
    cur = plsc.load_gather(tab_ref, [sd], mask=lastm)
    upd = jnp.maximum(cur, sv) if is_max else cur + sv
    plsc.store_scatter(tab_ref, [sd], upd, mask=lastm)


def _scores16(ox16, oy16):
    return 1.0 / (jnp.abs(ox16) + jnp.abs(oy16) + 0.001)


def _sca_body(dst_hbm, ox_hbm, oy_hbm, out_hbm,
              dstv, oxv, oyv, tabv, smaxv, accv, tmpv,
              stage_sh, smax_sh):
    cid = lax.axis_index("c")
    sid = lax.axis_index("s")
    wid = cid * NS + sid

    def zero_tab():
        @pl.loop(0, NP // L)
        def _(i):
            tabv[pl.ds(i * L, L)] = jnp.zeros((L,), jnp.float32)

    def combine(is_max):
        pltpu.sync_copy(tabv, stage_sh.at[sid])
        plsc.subcore_barrier()
        sl = pl.ds(sid * SEG, SEG)
        pltpu.sync_copy(stage_sh.at[0, sl], accv)
        for j in range(1, NS):
            pltpu.sync_copy(stage_sh.at[j, sl], tmpv)

            @pl.loop(0, SEG // L)
            def _(i):
                a = accv[pl.ds(i * L, L)]
                t = tmpv[pl.ds(i * L, L)]
                accv[pl.ds(i * L, L)] = jnp.maximum(a, t) if is_max else a + t

    zero_tab()

    @pl.loop(0, NCHUNK_A)
    def _(cc):
        base = wid * TILE_E + cc * SCH
        pltpu.sync_copy(dst_hbm.at[pl.ds(base, SCH)], dstv)
        pltpu.sync_copy(ox_hbm.at[pl.ds(base, SCH)], oxv)
        pltpu.sync_copy(oy_hbm.at[pl.ds(base, SCH)], oyv)

        @pl.loop(0, SCH // L)
        def _(i):
            s = pl.ds(i * L, L)
            _seg_rmw(tabv, dstv[s], _scores16(oxv[s], oyv[s]), True)

    combine(True)
    sl = pl.ds(sid * SEG, SEG)
    pltpu.sync_copy(accv, smax_sh.at[sl])
    pltpu.sync_copy(accv, out_hbm.at[cid, 0, sl])
    plsc.subcore_barrier()
    pltpu.sync_copy(smax_sh, smaxv)

    zero_tab()

    @pl.loop(0, NCHUNK_A)
    def _(cc):
        base = wid * TILE_E + cc * SCH
        pltpu.sync_copy(dst_hbm.at[pl.ds(base, SCH)], dstv)
        pltpu.sync_copy(ox_hbm.at[pl.ds(base, SCH)], oxv)
        pltpu.sync_copy(oy_hbm.at[pl.ds(base, SCH)], oyv)

        @pl.loop(0, SCH // L)
        def _(i):
            s = pl.ds(i * L, L)
            d16 = dstv[s]
            s16 = _scores16(oxv[s], oyv[s])
            mx16 = plsc.load_gather(smaxv, [d16])
            _seg_rmw(tabv, d16, jnp.exp(s16 - mx16), False)

    combine(False)
    pltpu.sync_copy(accv, out_hbm.at[cid, 1, sl])


def _sc_scalar(dst1d, ox, oy):
    mesh = plsc.VectorSubcoreMesh(core_axis_name="c", subcore_axis_name="s")
    f32 = jnp.float32
    fn = pl.kernel(
        _sca_body,
        out_type=jax.ShapeDtypeStruct((NC, 2, NP), f32),
        mesh=mesh,
        scratch_types=[
            pltpu.VMEM((SCH,), jnp.int32),
            pltpu.VMEM((SCH,), f32),
            pltpu.VMEM((SCH,), f32),
            pltpu.VMEM((NP,), f32),
            pltpu.VMEM((NP,), f32),
            pltpu.VMEM((SEG,), f32),
            pltpu.VMEM((SEG,), f32),
            pltpu.VMEM_SHARED((NS, NP), f32),
            pltpu.VMEM_SHARED((NP,), f32),
        ],
        compiler_params=pltpu.CompilerParams(needs_layout_passes=False),
    )
    return fn(dst1d, ox, oy)


def _t_body(sm0_ref, ss0_ref, sm1_ref, ss1_ref, out_ref):
    sm0, ss0 = sm0_ref[...], ss0_ref[...]
    sm1, ss1 = sm1_ref[...], ss1_ref[...]
    smg = jnp.maximum(sm0, sm1)
    ssg = ss0 * jnp.exp(sm0 - smg) + ss1 * jnp.exp(sm1 - smg)
    out_ref[...] = jnp.where(ssg > 0, smg + jnp.log(jnp.maximum(ssg, 1e-30)), 0.0)


def _t_pallas(sm0, ss0, sm1, ss1):
    return pl.pallas_call(
        _t_body,
        out_shape=jax.ShapeDtypeStruct((NP // D, D), jnp.float32),
    )(sm0, ss0, sm1, ss1)


def _scw_body(t_hbm, dst_hbm, ox_hbm, oy_hbm, out_hbm,
              tv, dstv, oxv, oyv, wv):
    cid = lax.axis_index("c")
    sid = lax.axis_index("s")
    wid = cid * NS + sid

    pltpu.sync_copy(t_hbm, tv)

    @pl.loop(0, NCHUNK_B)
    def _(c):
        ebase = wid * TILE_E + c * CH
        pltpu.sync_copy(dst_hbm.at[pl.ds(ebase, CH)], dstv)
        pltpu.sync_copy(ox_hbm.at[pl.ds(ebase, CH)], oxv)
        pltpu.sync_copy(oy_hbm.at[pl.ds(ebase, CH)], oyv)

        @pl.loop(0, CH // L)
        def _(i):
            s = pl.ds(i * L, L)
            d16 = dstv[s]
            s16 = _scores16(oxv[s], oyv[s])
            t16 = plsc.load_gather(tv, [d16])
            wv[s] = jnp.exp(s16 - t16)

        pltpu.sync_copy(wv, out_hbm.at[pl.ds(ebase, CH)])


def _sc_weights(t, dst1d, ox, oy):
    mesh = plsc.VectorSubcoreMesh(core_axis_name="c", subcore_axis_name="s")
    f32 = jnp.float32
    fn = pl.kernel(
        _scw_body,
        out_type=jax.ShapeDtypeStruct((E,), f32),
        mesh=mesh,
        scratch_types=[
            pltpu.VMEM((NP,), f32),
            pltpu.VMEM((CH,), jnp.int32),
            pltpu.VMEM((CH,), f32),
            pltpu.VMEM((CH,), f32),
            pltpu.VMEM((CH,), f32),
        ],
        compiler_params=pltpu.CompilerParams(needs_layout_passes=False),
    )
    return fn(t, dst1d, ox, oy)


def _scb_body(src3_hbm, dst3_hbm, feat_hbm, kw_hbm, out_hbm,
              srcv2, dstv2, fb0, fb1, kb,
              xagg_sh, gsem0, gsem1, ksem, ssem):
    cid = lax.axis_index("c")
    sid = lax.axis_index("s")
    wid = cid * NS + sid

    @pl.loop(0, EB)
    def _(e):
        for q in range(D // L):
            fb0[e, pl.ds(q * L, L)] = jnp.zeros((L,), jnp.float32)
    for t in range(SEG // EB):
        pltpu.sync_copy(fb0, xagg_sh.at[pl.ds(sid * SEG + t * EB, EB)])
    plsc.subcore_barrier()

    fbs = (fb0, fb1)
    gsems = (gsem0, gsem1)

    @pl.loop(0, NCHUNK_B)
    def _(c):
        ebase = wid * TILE_E + c * CH
        pltpu.sync_copy(src3_hbm.at[wid, c], srcv2)
        pltpu.sync_copy(dst3_hbm.at[wid, c], dstv2)

        def issue_g(j, b):
            pltpu.async_copy(feat_hbm.at[srcv2.at[j]], fbs[b], gsems[b])

        def wait_g(j, b):
            pltpu.make_async_copy(feat_hbm.at[srcv2.at[j]], fbs[b],
                                  gsems[b]).wait()

        def wait_scat(j):
            pltpu.make_async_copy(kb, xagg_sh.at[dstv2.at[j]], ssem).wait()

        def block(j, b):
            @pl.when(j + 1 < CB)
            def _():
                issue_g(j + 1, 1 - b)

            @pl.when(j >= 1)
            def _():
                wait_scat(j - 1)
            kwsl = kw_hbm.at[pl.ds(ebase + j * EB, EB)]
            pltpu.async_copy(kwsl, kb, ksem)
            wait_g(j, b)
            pltpu.make_async_copy(kwsl, kb, ksem).wait()
            fb = fbs[b]

            @pl.loop(0, EB, unroll=8)
            def _(e):
                for q in range(D // L):
                    cc = pl.ds(q * L, L)
                    kb[e, cc] = kb[e, cc] * fb[e, cc]

            pltpu.async_copy(kb, xagg_sh.at[dstv2.at[j]], ssem, add=True)

        issue_g(0, 0)

        @pl.loop(0, CB // 2)
        def _(jp):
            for b in range(2):
                block(jp * 2 + b, b)

        block(CB - 1, 0)
        wait_scat(CB - 1)

    plsc.subcore_barrier()
    sl = pl.ds(sid * SEG, SEG)
    pltpu.sync_copy(xagg_sh.at[sl], out_hbm.at[cid, sl])


def _sc_vector(src3d, dst3d, feat, kw):
    mesh = plsc.VectorSubcoreMesh(core_axis_name="c", subcore_axis_name="s")
    f32 = jnp.float32
    i32 = jnp.int32
    fn = pl.kernel(
        _scb_body,
        out_type=jax.ShapeDtypeStruct((NC, NP, D), f32),
        mesh=mesh,
        scratch_types=[
            pltpu.VMEM((CB, EB), i32),
            pltpu.VMEM((CB, EB), i32),
            pltpu.VMEM((EB, D), f32),
            pltpu.VMEM((EB, D), f32),
            pltpu.VMEM((EB, D), f32),
            pltpu.VMEM_SHARED((NP, D), f32),
            pltpu.SemaphoreType.DMA,
            pltpu.SemaphoreType.DMA,
            pltpu.SemaphoreType.DMA,
            pltpu.SemaphoreType.DMA,
        ],
        compiler_params=pltpu.CompilerParams(needs_layout_passes=False),
    )
    return fn(src3d, dst3d, feat, kw)


def _mlp_body(xagg_ref, feat_ref, sW0_ref, sb0_ref, sW1_ref, sb1_ref,
              mW0_ref, mb0_ref, mW1_ref, mb1_ref, mW2_ref, mb2_ref,
              mW3_ref, mb3_ref, out_ref):
    xa = xagg_ref[0] + xagg_ref[1]
    feat = feat_ref[...]
    hs = _elu(jnp.dot(feat, sW0_ref[...], preferred_element_type=jnp.float32) + sb0_ref[...])
    hs = jnp.dot(hs, sW1_ref[...], preferred_element_type=jnp.float32) + sb1_ref[...]
    h = _elu(jnp.dot(xa, mW0_ref[0:D], preferred_element_type=jnp.float32)
             + jnp.dot(hs, mW0_ref[D:2 * D], preferred_element_type=jnp.float32)
             + mb0_ref[...])
    h = _elu(jnp.dot(h, mW1_ref[...], preferred_element_type=jnp.float32) + mb1_ref[...])
    h = _elu(jnp.dot(h, mW2_ref[...], preferred_element_type=jnp.float32) + mb2_ref[...])
    out_ref[...] = jnp.dot(h, mW3_ref[...], preferred_element_type=jnp.float32) + mb3_ref[...]


def _mlp_pallas(xaggp, feat, sW0, sb0, sW1, sb1, mW0, mb0, mW1, mb1, mW2, mb2, mW3, mb3):
    row = lambda i: (i, 0)
    rep2 = lambda i: (0, 0)
    rep3 = lambda i: (0, i, 0)
    return pl.pallas_call(
        _mlp_body,
        grid=(N // BLK_N,),
        in_specs=[
            pl.BlockSpec((2, BLK_N, D), rep3),
            pl.BlockSpec((BLK_N, D), row),
            pl.BlockSpec((D, D), rep2), pl.BlockSpec((1, D), rep2),
            pl.BlockSpec((D, D), rep2), pl.BlockSpec((1, D), rep2),
            pl.BlockSpec((2 * D, D), rep2), pl.BlockSpec((1, D), rep2),
            pl.BlockSpec((D, D), rep2), pl.BlockSpec((1, D), rep2),
            pl.BlockSpec((D, D), rep2), pl.BlockSpec((1, D), rep2),
            pl.BlockSpec((D, D), rep2), pl.BlockSpec((1, D), rep2),
        ],
        out_specs=pl.BlockSpec((BLK_N, D), row),
        out_shape=jax.ShapeDtypeStruct((N, D), jnp.float32),
    )(xaggp, feat, sW0, sb0.reshape(1, D), sW1, sb1.reshape(1, D),
      mW0, mb0.reshape(1, D), mW1, mb1.reshape(1, D), mW2, mb2.reshape(1, D),
      mW3, mb3.reshape(1, D))


def kernel(feat, edge_index, offsets, kW0, kb0, kW1, kb1, sW0, sb0, sW1, sb1,
           mW0, mb0, mW1, mb1, mW2, mb2, mW3, mb3):
    src = edge_index[0]
    dst = edge_index[1]
    ox = offsets[:, 0]
    oy = offsets[:, 1]
    src3d = src.reshape(NC * NS, NCHUNK_B, CB, EB)
    dst3d = dst.reshape(NC * NS, NCHUNK_B, CB, EB)

    sums = _sc_scalar(dst, ox, oy)
    shp = (NP // D, D)
    t = _t_pallas(sums[0, 0].reshape(shp), sums[0, 1].reshape(shp),
                  sums[1, 0].reshape(shp), sums[1, 1].reshape(shp))
    w = _sc_weights(t.reshape(NP), dst, ox, oy)
    kw = _kw_pallas(offsets, w, kW0, kb0, kW1, kb1)
    xaggp = _sc_vector(src3d, dst3d, feat, kw)

    return _mlp_pallas(xaggp, feat, sW0, sb0, sW1, sb1,
                       mW0, mb0, mW1, mb1, mW2, mb2, mW3, mb3)


# speedup vs baseline: 1.1225x; 1.1225x over previous
import jax
import jax.numpy as jnp
from jax import lax
from jax.experimental import pallas as pl
from jax.experimental.pallas import tpu as pltpu
from jax.experimental.pallas import tpu_sc as plsc

N = 10000
NP = 10240
E = 320000
D = 128

NC = 2
NS = 16
L = 16

EB = 80
CH = 2000
CB = CH // EB
SCH = 2000
TILE_E = E // (NC * NS)
NCHUNK_A = TILE_E // SCH
NCHUNK_B = TILE_E // CH
SEG = NP // NS

BLK_E = 2000
BLK_N = 1000


def _elu(x):
    return jnp.where(x > 0, x, jnp.exp(jnp.minimum(x, 0.0)) - 1.0)


def _kw_body(off_ref, w_ref, kW0_ref, kb0_ref, kW1_ref, kb1_ref, out_ref):
    off = off_ref[...]
    w0 = kW0_ref[...]
    h = off[:, 0:1] * w0[0:1, :] + off[:, 1:2] * w0[1:2, :] + kb0_ref[...]
    h = _elu(h)
    kw = jnp.dot(h, kW1_ref[...], preferred_element_type=jnp.float32) + kb1_ref[...]
    out_ref[...] = kw * w_ref[...]


def _kw_pallas(offsets, w, kW0, kb0, kW1, kb1):
    return pl.pallas_call(
        _kw_body,
        grid=(E // BLK_E,),
        in_specs=[
            pl.BlockSpec((BLK_E, 2), lambda i: (i, 0)),
            pl.BlockSpec((BLK_E, 1), lambda i: (i, 0)),
            pl.BlockSpec((2, D), lambda i: (0, 0)),
            pl.BlockSpec((1, D), lambda i: (0, 0)),
            pl.BlockSpec((D, D), lambda i: (0, 0)),
            pl.BlockSpec((1, D), lambda i: (0, 0)),
        ],
        out_specs=pl.BlockSpec((BLK_E, D), lambda i: (i, 0)),
        out_shape=jax.ShapeDtypeStruct((E, D), jnp.float32),
    )(offsets, w.reshape(E, 1), kW0, kb0.reshape(1, D), kW1, kb1.reshape(1, D))


def _g16(x, idx):
    return lax.gather(
        x, idx[:, None],
        lax.GatherDimensionNumbers(
            offset_dims=(), collapsed_slice_dims=(0,), start_index_map=(0,)),
        (1,), mode=lax.GatherScatterMode.PROMISE_IN_BOUNDS)


def _seg_rmw(tab_ref, d16, v16, is_max):
    iota = lax.iota(jnp.int32, L)
    sd, sv = plsc.sort_key_val(d16, v16)
    for k in (1, 2, 4, 8):
        idx = jnp.maximum(iota - k, 0)
        sh_v = _g16(sv, idx)
        sh_d = _g16(sd, idx)
        m = (sh_d == sd) & (iota >= k)
        if is_max:
            sv = jnp.where(m, jnp.maximum(sv, sh_v), sv)
        else:
            sv = jnp.where(m, sv + sh_v, sv)
    nxt = _g16(sd, jnp.minimum(iota + 1, L - 1))
    lastm = (nxt != sd) | (iota == L - 1)
    cur = plsc.load_gather(tab_ref, [sd], mask=lastm)
    upd = jnp.maximum(cur, sv) if is_max else cur + sv
    plsc.store_scatter(tab_ref, [sd], upd, mask=lastm)


def _scores16(ox16, oy16):
    return 1.0 / (jnp.abs(ox16) + jnp.abs(oy16) + 0.001)


def _sca_body(dst_hbm, ox_hbm, oy_hbm, out_hbm,
              dstv, oxv, oyv, tabv, smaxv, accv, tmpv,
              stage_sh, smax_sh):
    cid = lax.axis_index("c")
    sid = lax.axis_index("s")
    wid = cid * NS + sid

    def zero_tab():
        @pl.loop(0, NP // L)
        def _(i):
            tabv[pl.ds(i * L, L)] = jnp.zeros((L,), jnp.float32)

    def combine(is_max):
        pltpu.sync_copy(tabv, stage_sh.at[sid])
        plsc.subcore_barrier()
        sl = pl.ds(sid * SEG, SEG)
        pltpu.sync_copy(stage_sh.at[0, sl], accv)
        for j in range(1, NS):
            pltpu.sync_copy(stage_sh.at[j, sl], tmpv)

            @pl.loop(0, SEG // L)
            def _(i):
                a = accv[pl.ds(i * L, L)]
                t = tmpv[pl.ds(i * L, L)]
                accv[pl.ds(i * L, L)] = jnp.maximum(a, t) if is_max else a + t

    zero_tab()

    @pl.loop(0, NCHUNK_A)
    def _(cc):
        base = wid * TILE_E + cc * SCH
        pltpu.sync_copy(dst_hbm.at[pl.ds(base, SCH)], dstv)
        pltpu.sync_copy(ox_hbm.at[pl.ds(base, SCH)], oxv)
        pltpu.sync_copy(oy_hbm.at[pl.ds(base, SCH)], oyv)

        @pl.loop(0, SCH // L)
        def _(i):
            s = pl.ds(i * L, L)
            _seg_rmw(tabv, dstv[s], _scores16(oxv[s], oyv[s]), True)

    combine(True)
    sl = pl.ds(sid * SEG, SEG)
    pltpu.sync_copy(accv, smax_sh.at[sl])
    pltpu.sync_copy(accv, out_hbm.at[cid, 0, sl])
    plsc.subcore_barrier()
    pltpu.sync_copy(smax_sh, smaxv)

    zero_tab()

    @pl.loop(0, NCHUNK_A)
    def _(cc):
        base = wid * TILE_E + cc * SCH
        pltpu.sync_copy(dst_hbm.at[pl.ds(base, SCH)], dstv)
        pltpu.sync_copy(ox_hbm.at[pl.ds(base, SCH)], oxv)
        pltpu.sync_copy(oy_hbm.at[pl.ds(base, SCH)], oyv)

        @pl.loop(0, SCH // L)
        def _(i):
            s = pl.ds(i * L, L)
            d16 = dstv[s]
            s16 = _scores16(oxv[s], oyv[s])
            mx16 = plsc.load_gather(smaxv, [d16])
            _seg_rmw(tabv, d16, jnp.exp(s16 - mx16), False)

    combine(False)
    pltpu.sync_copy(accv, out_hbm.at[cid, 1, sl])


def _sc_scalar(dst1d, ox, oy):
    mesh = plsc.VectorSubcoreMesh(core_axis_name="c", subcore_axis_name="s")
    f32 = jnp.float32
    fn = pl.kernel(
        _sca_body,
        out_type=jax.ShapeDtypeStruct((NC, 2, NP), f32),
        mesh=mesh,
        scratch_types=[
            pltpu.VMEM((SCH,), jnp.int32),
            pltpu.VMEM((SCH,), f32),
            pltpu.VMEM((SCH,), f32),
            pltpu.VMEM((NP,), f32),
            pltpu.VMEM((NP,), f32),
            pltpu.VMEM((SEG,), f32),
            pltpu.VMEM((SEG,), f32),
            pltpu.VMEM_SHARED((NS, NP), f32),
            pltpu.VMEM_SHARED((NP,), f32),
        ],
        compiler_params=pltpu.CompilerParams(needs_layout_passes=False),
    )
    return fn(dst1d, ox, oy)


def _t_body(sm0_ref, ss0_ref, sm1_ref, ss1_ref, out_ref):
    sm0, ss0 = sm0_ref[...], ss0_ref[...]
    sm1, ss1 = sm1_ref[...], ss1_ref[...]
    smg = jnp.maximum(sm0, sm1)
    ssg = ss0 * jnp.exp(sm0 - smg) + ss1 * jnp.exp(sm1 - smg)
    out_ref[...] = jnp.where(ssg > 0, smg + jnp.log(jnp.maximum(ssg, 1e-30)), 0.0)


def _t_pallas(sm0, ss0, sm1, ss1):
    return pl.pallas_call(
        _t_body,
        out_shape=jax.ShapeDtypeStruct((NP // D, D), jnp.float32),
    )(sm0, ss0, sm1, ss1)


def _scw_body(t_hbm, dst_hbm, ox_hbm, oy_hbm, out_hbm,
              tv, dstv, oxv, oyv, wv):
    cid = lax.axis_index("c")
    sid = lax.axis_index("s")
    wid = cid * NS + sid

    pltpu.sync_copy(t_hbm, tv)

    @pl.loop(0, NCHUNK_B)
    def _(c):
        ebase = wid * TILE_E + c * CH
        pltpu.sync_copy(dst_hbm.at[pl.ds(ebase, CH)], dstv)
        pltpu.sync_copy(ox_hbm.at[pl.ds(ebase, CH)], oxv)
        pltpu.sync_copy(oy_hbm.at[pl.ds(ebase, CH)], oyv)

        @pl.loop(0, CH // L)
        def _(i):
            s = pl.ds(i * L, L)
            d16 = dstv[s]
            s16 = _scores16(oxv[s], oyv[s])
            t16 = plsc.load_gather(tv, [d16])
            wv[s] = jnp.exp(s16 - t16)

        pltpu.sync_copy(wv, out_hbm.at[pl.ds(ebase, CH)])


def _sc_weights(t, dst1d, ox, oy):
    mesh = plsc.VectorSubcoreMesh(core_axis_name="c", subcore_axis_name="s")
    f32 = jnp.float32
    fn = pl.kernel(
        _scw_body,
        out_type=jax.ShapeDtypeStruct((E,), f32),
        mesh=mesh,
        scratch_types=[
            pltpu.VMEM((NP,), f32),
            pltpu.VMEM((CH,), jnp.int32),
            pltpu.VMEM((CH,), f32),
            pltpu.VMEM((CH,), f32),
            pltpu.VMEM((CH,), f32),
        ],
        compiler_params=pltpu.CompilerParams(needs_layout_passes=False),
    )
    return fn(t, dst1d, ox, oy)


def _scb_body(src3_hbm, dst3_hbm, feat_hbm, kw_hbm, out_hbm,
              srcv2, dstv2, fb0, fb1, kb,
              xagg_sh, gsem0, gsem1, ksem, ssem):
    cid = lax.axis_index("c")
    sid = lax.axis_index("s")
    wid = cid * NS + sid

    @pl.loop(0, EB)
    def _(e):
        for q in range(D // L):
            fb0[e, pl.ds(q * L, L)] = jnp.zeros((L,), jnp.float32)
    for t in range(SEG // EB):
        pltpu.sync_copy(fb0, xagg_sh.at[pl.ds(sid * SEG + t * EB, EB)])
    plsc.subcore_barrier()

    fbs = (fb0, fb1)
    gsems = (gsem0, gsem1)

    @pl.loop(0, NCHUNK_B)
    def _(c):
        ebase = wid * TILE_E + c * CH
        pltpu.sync_copy(src3_hbm.at[wid, c], srcv2)
        pltpu.sync_copy(dst3_hbm.at[wid, c], dstv2)

        def issue_g(j, b):
            pltpu.async_copy(feat_hbm.at[srcv2.at[j]], fbs[b], gsems[b])

        def wait_g(j, b):
            pltpu.make_async_copy(feat_hbm.at[srcv2.at[j]], fbs[b],
                                  gsems[b]).wait()

        def wait_scat(j):
            pltpu.make_async_copy(kb, xagg_sh.at[dstv2.at[j]], ssem).wait()

        def block(j, b):
            @pl.when(j + 1 < CB)
            def _():
                issue_g(j + 1, 1 - b)

            @pl.when(j >= 1)
            def _():
                wait_scat(j - 1)
            kwsl = kw_hbm.at[pl.ds(ebase + j * EB, EB)]
            pltpu.async_copy(kwsl, kb, ksem)
            wait_g(j, b)
            pltpu.make_async_copy(kwsl, kb, ksem).wait()
            fb = fbs[b]

            @pl.loop(0, EB // L)
            def _(e16):
                for lane in range(L):
                    e = e16 * L + lane
                    for q in range(D // L):
                        cc = pl.ds(q * L, L)
                        kb[e, cc] = kb[e, cc] * fb[e, cc]

            pltpu.async_copy(kb, xagg_sh.at[dstv2.at[j]], ssem, add=True)

        issue_g(0, 0)

        @pl.loop(0, CB // 2)
        def _(jp):
            for b in range(2):
                block(jp * 2 + b, b)

        block(CB - 1, 0)
        wait_scat(CB - 1)

    plsc.subcore_barrier()
    sl = pl.ds(sid * SEG, SEG)
    pltpu.sync_copy(xagg_sh.at[sl], out_hbm.at[cid, sl])


def _sc_vector(src3d, dst3d, feat, kw):
    mesh = plsc.VectorSubcoreMesh(core_axis_name="c", subcore_axis_name="s")
    f32 = jnp.float32
    i32 = jnp.int32
    fn = pl.kernel(
        _scb_body,
        out_type=jax.ShapeDtypeStruct((NC, NP, D), f32),
        mesh=mesh,
        scratch_types=[
            pltpu.VMEM((CB, EB), i32),
            pltpu.VMEM((CB, EB), i32),
            pltpu.VMEM((EB, D), f32),
            pltpu.VMEM((EB, D), f32),
            pltpu.VMEM((EB, D), f32),
            pltpu.VMEM_SHARED((NP, D), f32),
            pltpu.SemaphoreType.DMA,
            pltpu.SemaphoreType.DMA,
            pltpu.SemaphoreType.DMA,
            pltpu.SemaphoreType.DMA,
        ],
        compiler_params=pltpu.CompilerParams(needs_layout_passes=False),
    )
    return fn(src3d, dst3d, feat, kw)


def _mlp_body(xagg_ref, feat_ref, sW0_ref, sb0_ref, sW1_ref, sb1_ref,
              mW0_ref, mb0_ref, mW1_ref, mb1_ref, mW2_ref, mb2_ref,
              mW3_ref, mb3_ref, out_ref):
    xa = xagg_ref[0] + xagg_ref[1]
    feat = feat_ref[...]
    hs = _elu(jnp.dot(feat, sW0_ref[...], preferred_element_type=jnp.float32) + sb0_ref[...])
    hs = jnp.dot(hs, sW1_ref[...], preferred_element_type=jnp.float32) + sb1_ref[...]
    h = _elu(jnp.dot(xa, mW0_ref[0:D], preferred_element_type=jnp.float32)
             + jnp.dot(hs, mW0_ref[D:2 * D], preferred_element_type=jnp.float32)
             + mb0_ref[...])
    h = _elu(jnp.dot(h, mW1_ref[...], preferred_element_type=jnp.float32) + mb1_ref[...])
    h = _elu(jnp.dot(h, mW2_ref[...], preferred_element_type=jnp.float32) + mb2_ref[...])
    out_ref[...] = jnp.dot(h, mW3_ref[...], preferred_element_type=jnp.float32) + mb3_ref[...]


def _mlp_pallas(xaggp, feat, sW0, sb0, sW1, sb1, mW0, mb0, mW1, mb1, mW2, mb2, mW3, mb3):
    row = lambda i: (i, 0)
    rep2 = lambda i: (0, 0)
    rep3 = lambda i: (0, i, 0)
    return pl.pallas_call(
        _mlp_body,
        grid=(N // BLK_N,),
        in_specs=[
            pl.BlockSpec((2, BLK_N, D), rep3),
            pl.BlockSpec((BLK_N, D), row),
            pl.BlockSpec((D, D), rep2), pl.BlockSpec((1, D), rep2),
            pl.BlockSpec((D, D), rep2), pl.BlockSpec((1, D), rep2),
            pl.BlockSpec((2 * D, D), rep2), pl.BlockSpec((1, D), rep2),
            pl.BlockSpec((D, D), rep2), pl.BlockSpec((1, D), rep2),
            pl.BlockSpec((D, D), rep2), pl.BlockSpec((1, D), rep2),
            pl.BlockSpec((D, D), rep2), pl.BlockSpec((1, D), rep2),
        ],
        out_specs=pl.BlockSpec((BLK_N, D), row),
        out_shape=jax.ShapeDtypeStruct((N, D), jnp.float32),
    )(xaggp, feat, sW0, sb0.reshape(1, D), sW1, sb1.reshape(1, D),
      mW0, mb0.reshape(1, D), mW1, mb1.reshape(1, D), mW2, mb2.reshape(1, D),
      mW3, mb3.reshape(1, D))


def kernel(feat, edge_index, offsets, kW0, kb0, kW1, kb1, sW0, sb0, sW1, sb1,
           mW0, mb0, mW1, mb1, mW2, mb2, mW3, mb3):
    src = edge_index[0]
    dst = edge_index[1]
    ox = offsets[:, 0]
    oy = offsets[:, 1]
    src3d = src.reshape(NC * NS, NCHUNK_B, CB, EB)
    dst3d = dst.reshape(NC * NS, NCHUNK_B, CB, EB)

    sums = _sc_scalar(dst, ox, oy)
    shp = (NP // D, D)
    t = _t_pallas(sums[0, 0].reshape(shp), sums[0, 1].reshape(shp),
                  sums[1, 0].reshape(shp), sums[1, 1].reshape(shp))
    w = _sc_weights(t.reshape(NP), dst, ox, oy)
    kw = _kw_pallas(offsets, w, kW0, kb0, kW1, kb1)
    xaggp = _sc_vector(src3d, dst3d, feat, kw)

    return _mlp_pallas(xaggp, feat, sW0, sb0, sW1, sb1,
                       mW0, mb0, mW1, mb1, mW2, mb2, mW3, mb3)


# speedup vs baseline: 1.4950x; 1.3319x over previous
import jax
import jax.numpy as jnp
from jax import lax
from jax.experimental import pallas as pl
from jax.experimental.pallas import tpu as pltpu
from jax.experimental.pallas import tpu_sc as plsc

N = 10000
NP = 10240
E = 320000
D = 128

NC = 2
NS = 16
L = 16

EB = 80
CH = 2000
CB = CH // EB
SCH = 2000
TILE_E = E // (NC * NS)
NCHUNK_A = TILE_E // SCH
NCHUNK_B = TILE_E // CH
SEG = NP // NS

BLK_E = 2000
BLK_N = 1000


def _elu(x):
    return jnp.where(x > 0, x, jnp.exp(jnp.minimum(x, 0.0)) - 1.0)


def _kw_body(off_ref, sm0_ref, ss0_ref, sm1_ref, ss1_ref,
             kW0_ref, kb0_ref, kW1_ref, kb1_ref, out_ref, t_ref):
    off = off_ref[...]
    w0 = kW0_ref[...]
    h = off[:, 0:1] * w0[0:1, :] + off[:, 1:2] * w0[1:2, :] + kb0_ref[...]
    h = _elu(h)
    out_ref[...] = jnp.dot(h, kW1_ref[...], preferred_element_type=jnp.float32) + kb1_ref[...]

    @pl.when(pl.program_id(0) == 0)
    def _():
        sm0, ss0 = sm0_ref[...], ss0_ref[...]
        sm1, ss1 = sm1_ref[...], ss1_ref[...]
        smg = jnp.maximum(sm0, sm1)
        ssg = ss0 * jnp.exp(sm0 - smg) + ss1 * jnp.exp(sm1 - smg)
        t_ref[...] = jnp.where(ssg > 0, smg + jnp.log(jnp.maximum(ssg, 1e-30)), 0.0)


def _kw_pallas(offsets, sums, kW0, kb0, kW1, kb1):
    shp = (NP // D, D)
    rep2 = lambda i: (0, 0)
    kw, t = pl.pallas_call(
        _kw_body,
        grid=(E // BLK_E,),
        in_specs=[
            pl.BlockSpec((BLK_E, 2), lambda i: (i, 0)),
            pl.BlockSpec(shp, rep2), pl.BlockSpec(shp, rep2),
            pl.BlockSpec(shp, rep2), pl.BlockSpec(shp, rep2),
            pl.BlockSpec((2, D), rep2),
            pl.BlockSpec((1, D), rep2),
            pl.BlockSpec((D, D), rep2),
            pl.BlockSpec((1, D), rep2),
        ],
        out_specs=[pl.BlockSpec((BLK_E, D), lambda i: (i, 0)),
                   pl.BlockSpec(shp, rep2)],
        out_shape=[jax.ShapeDtypeStruct((E, D), jnp.float32),
                   jax.ShapeDtypeStruct(shp, jnp.float32)],
    )(offsets, sums[0, 0].reshape(shp), sums[0, 1].reshape(shp),
      sums[1, 0].reshape(shp), sums[1, 1].reshape(shp),
      kW0, kb0.reshape(1, D), kW1, kb1.reshape(1, D))
    return kw, t


def _g16(x, idx):
    return lax.gather(
        x, idx[:, None],
        lax.GatherDimensionNumbers(
            offset_dims=(), collapsed_slice_dims=(0,), start_index_map=(0,)),
        (1,), mode=lax.GatherScatterMode.PROMISE_IN_BOUNDS)


def _seg_rmw(tab_ref, d16, v16, is_max):
    iota = lax.iota(jnp.int32, L)
    sd, sv = plsc.sort_key_val(d16, v16)
    for k in (1, 2, 4, 8):
        idx = jnp.maximum(iota - k, 0)
        sh_v = _g16(sv, idx)
        sh_d = _g16(sd, idx)
        m = (sh_d == sd) & (iota >= k)
        if is_max:
            sv = jnp.where(m, jnp.maximum(sv, sh_v), sv)
        else:
            sv = jnp.where(m, sv + sh_v, sv)
    nxt = _g16(sd, jnp.minimum(iota + 1, L - 1))
    lastm = (nxt != sd) | (iota == L - 1)
    cur = plsc.load_gather(tab_ref, [sd], mask=lastm)
    upd = jnp.maximum(cur, sv) if is_max else cur + sv
    plsc.store_scatter(tab_ref, [sd], upd, mask=lastm)


def _scores16(ox16, oy16):
    return 1.0 / (jnp.abs(ox16) + jnp.abs(oy16) + 0.001)


def _sca_body(dst_hbm, ox_hbm, oy_hbm, out_hbm,
              dstv, oxv, oyv, tabv, smaxv, accv, tmpv,
              stage_sh, smax_sh):
    cid = lax.axis_index("c")
    sid = lax.axis_index("s")
    wid = cid * NS + sid

    def zero_tab():
        @pl.loop(0, NP // L)
        def _(i):
            tabv[pl.ds(i * L, L)] = jnp.zeros((L,), jnp.float32)

    def combine(is_max):
        pltpu.sync_copy(tabv, stage_sh.at[sid])
        plsc.subcore_barrier()
        sl = pl.ds(sid * SEG, SEG)
        pltpu.sync_copy(stage_sh.at[0, sl], accv)
        for j in range(1, NS):
            pltpu.sync_copy(stage_sh.at[j, sl], tmpv)

            @pl.loop(0, SEG // L)
            def _(i):
                a = accv[pl.ds(i * L, L)]
                t = tmpv[pl.ds(i * L, L)]
                accv[pl.ds(i * L, L)] = jnp.maximum(a, t) if is_max else a + t

    zero_tab()

    @pl.loop(0, NCHUNK_A)
    def _(cc):
        base = wid * TILE_E + cc * SCH
        pltpu.sync_copy(dst_hbm.at[pl.ds(base, SCH)], dstv)
        pltpu.sync_copy(ox_hbm.at[pl.ds(base, SCH)], oxv)
        pltpu.sync_copy(oy_hbm.at[pl.ds(base, SCH)], oyv)

        @pl.loop(0, SCH // L)
        def _(i):
            s = pl.ds(i * L, L)
            _seg_rmw(tabv, dstv[s], _scores16(oxv[s], oyv[s]), True)

    combine(True)
    sl = pl.ds(sid * SEG, SEG)
    pltpu.sync_copy(accv, smax_sh.at[sl])
    pltpu.sync_copy(accv, out_hbm.at[cid, 0, sl])
    plsc.subcore_barrier()
    pltpu.sync_copy(smax_sh, smaxv)

    zero_tab()

    @pl.loop(0, NCHUNK_A)
    def _(cc):
        base = wid * TILE_E + cc * SCH
        pltpu.sync_copy(dst_hbm.at[pl.ds(base, SCH)], dstv)
        pltpu.sync_copy(ox_hbm.at[pl.ds(base, SCH)], oxv)
        pltpu.sync_copy(oy_hbm.at[pl.ds(base, SCH)], oyv)

        @pl.loop(0, SCH // L)
        def _(i):
            s = pl.ds(i * L, L)
            d16 = dstv[s]
            s16 = _scores16(oxv[s], oyv[s])
            mx16 = plsc.load_gather(smaxv, [d16])
            _seg_rmw(tabv, d16, jnp.exp(s16 - mx16), False)

    combine(False)
    pltpu.sync_copy(accv, out_hbm.at[cid, 1, sl])


def _sc_scalar(dst1d, ox, oy):
    mesh = plsc.VectorSubcoreMesh(core_axis_name="c", subcore_axis_name="s")
    f32 = jnp.float32
    fn = pl.kernel(
        _sca_body,
        out_type=jax.ShapeDtypeStruct((NC, 2, NP), f32),
        mesh=mesh,
        scratch_types=[
            pltpu.VMEM((SCH,), jnp.int32),
            pltpu.VMEM((SCH,), f32),
            pltpu.VMEM((SCH,), f32),
            pltpu.VMEM((NP,), f32),
            pltpu.VMEM((NP,), f32),
            pltpu.VMEM((SEG,), f32),
            pltpu.VMEM((SEG,), f32),
            pltpu.VMEM_SHARED((NS, NP), f32),
            pltpu.VMEM_SHARED((NP,), f32),
        ],
        compiler_params=pltpu.CompilerParams(needs_layout_passes=False),
    )
    return fn(dst1d, ox, oy)


def _scw_body(t_hbm, dst_hbm, ox_hbm, oy_hbm, out_hbm,
              tv, dstv, oxv, oyv, wv):
    cid = lax.axis_index("c")
    sid = lax.axis_index("s")
    wid = cid * NS + sid

    pltpu.sync_copy(t_hbm, tv)

    @pl.loop(0, NCHUNK_B)
    def _(c):
        ebase = wid * TILE_E + c * CH
        pltpu.sync_copy(dst_hbm.at[pl.ds(ebase, CH)], dstv)
        pltpu.sync_copy(ox_hbm.at[pl.ds(ebase, CH)], oxv)
        pltpu.sync_copy(oy_hbm.at[pl.ds(ebase, CH)], oyv)

        @pl.loop(0, CH // L)
        def _(i):
            s = pl.ds(i * L, L)
            d16 = dstv[s]
            s16 = _scores16(oxv[s], oyv[s])
            t16 = plsc.load_gather(tv, [d16])
            wv[s] = jnp.exp(s16 - t16)

        pltpu.sync_copy(wv, out_hbm.at[pl.ds(ebase, CH)])


def _sc_weights(t, dst1d, ox, oy):
    mesh = plsc.VectorSubcoreMesh(core_axis_name="c", subcore_axis_name="s")
    f32 = jnp.float32
    fn = pl.kernel(
        _scw_body,
        out_type=jax.ShapeDtypeStruct((E,), f32),
        mesh=mesh,
        scratch_types=[
            pltpu.VMEM((NP,), f32),
            pltpu.VMEM((CH,), jnp.int32),
            pltpu.VMEM((CH,), f32),
            pltpu.VMEM((CH,), f32),
            pltpu.VMEM((CH,), f32),
        ],
        compiler_params=pltpu.CompilerParams(needs_layout_passes=False),
    )
    return fn(t, dst1d, ox, oy)


def _scb_body(w_hbm, src3_hbm, dst3_hbm, feat_hbm, kw_hbm, out_hbm,
              wv, srcv2, dstv2, fb0, fb1, kb,
              xagg_sh, gsem0, gsem1, ksem, ssem):
    cid = lax.axis_index("c")
    sid = lax.axis_index("s")
    wid = cid * NS + sid

    @pl.loop(0, EB)
    def _(e):
        for q in range(D // L):
            fb0[e, pl.ds(q * L, L)] = jnp.zeros((L,), jnp.float32)
    for t in range(SEG // EB):
        pltpu.sync_copy(fb0, xagg_sh.at[pl.ds(sid * SEG + t * EB, EB)])
    plsc.subcore_barrier()

    fbs = (fb0, fb1)
    gsems = (gsem0, gsem1)

    @pl.loop(0, NCHUNK_B)
    def _(c):
        ebase = wid * TILE_E + c * CH
        pltpu.sync_copy(w_hbm.at[pl.ds(ebase, CH)], wv)
        pltpu.sync_copy(src3_hbm.at[wid, c], srcv2)
        pltpu.sync_copy(dst3_hbm.at[wid, c], dstv2)

        def issue_g(j, b):
            pltpu.async_copy(feat_hbm.at[srcv2.at[j]], fbs[b], gsems[b])

        def wait_g(j, b):
            pltpu.make_async_copy(feat_hbm.at[srcv2.at[j]], fbs[b],
                                  gsems[b]).wait()

        def wait_scat(j):
            pltpu.make_async_copy(kb, xagg_sh.at[dstv2.at[j]], ssem).wait()

        def block(j, b):
            @pl.when(j + 1 < CB)
            def _():
                issue_g(j + 1, 1 - b)

            @pl.when(j >= 1)
            def _():
                wait_scat(j - 1)
            kwsl = kw_hbm.at[pl.ds(ebase + j * EB, EB)]
            pltpu.async_copy(kwsl, kb, ksem)
            wait_g(j, b)
            pltpu.make_async_copy(kwsl, kb, ksem).wait()
            fb = fbs[b]

            @pl.loop(0, EB // L)
            def _(e16):
                w16 = wv[pl.ds(j * EB + e16 * L, L)]
                for lane in range(L):
                    wb = _g16(w16, jnp.full((L,), lane, jnp.int32))
                    e = e16 * L + lane
                    for q in range(D // L):
                        cc = pl.ds(q * L, L)
                        kb[e, cc] = kb[e, cc] * fb[e, cc] * wb

            pltpu.async_copy(kb, xagg_sh.at[dstv2.at[j]], ssem, add=True)

        issue_g(0, 0)

        @pl.loop(0, CB // 2)
        def _(jp):
            for b in range(2):
                block(jp * 2 + b, b)

        block(CB - 1, 0)
        wait_scat(CB - 1)

    plsc.subcore_barrier()
    sl = pl.ds(sid * SEG, SEG)
    pltpu.sync_copy(xagg_sh.at[sl], out_hbm.at[cid, sl])


def _sc_vector(w, src3d, dst3d, feat, kw):
    mesh = plsc.VectorSubcoreMesh(core_axis_name="c", subcore_axis_name="s")
    f32 = jnp.float32
    i32 = jnp.int32
    fn = pl.kernel(
        _scb_body,
        out_type=jax.ShapeDtypeStruct((NC, NP, D), f32),
        mesh=mesh,
        scratch_types=[
            pltpu.VMEM((CH,), f32),
            pltpu.VMEM((CB, EB), i32),
            pltpu.VMEM((CB, EB), i32),
            pltpu.VMEM((EB, D), f32),
            pltpu.VMEM((EB, D), f32),
            pltpu.VMEM((EB, D), f32),
            pltpu.VMEM_SHARED((NP, D), f32),
            pltpu.SemaphoreType.DMA,
            pltpu.SemaphoreType.DMA,
            pltpu.SemaphoreType.DMA,
            pltpu.SemaphoreType.DMA,
        ],
        compiler_params=pltpu.CompilerParams(needs_layout_passes=False),
    )
    return fn(w, src3d, dst3d, feat, kw)


def _mlp_body(xagg_ref, feat_ref, sW0_ref, sb0_ref, sW1_ref, sb1_ref,
              mW0_ref, mb0_ref, mW1_ref, mb1_ref, mW2_ref, mb2_ref,
              mW3_ref, mb3_ref, out_ref):
    xa = xagg_ref[0] + xagg_ref[1]
    feat = feat_ref[...]
    hs = _elu(jnp.dot(feat, sW0_ref[...], preferred_element_type=jnp.float32) + sb0_ref[...])
    hs = jnp.dot(hs, sW1_ref[...], preferred_element_type=jnp.float32) + sb1_ref[...]
    h = _elu(jnp.dot(xa, mW0_ref[0:D], preferred_element_type=jnp.float32)
             + jnp.dot(hs, mW0_ref[D:2 * D], preferred_element_type=jnp.float32)
             + mb0_ref[...])
    h = _elu(jnp.dot(h, mW1_ref[...], preferred_element_type=jnp.float32) + mb1_ref[...])
    h = _elu(jnp.dot(h, mW2_ref[...], preferred_element_type=jnp.float32) + mb2_ref[...])
    out_ref[...] = jnp.dot(h, mW3_ref[...], preferred_element_type=jnp.float32) + mb3_ref[...]


def _mlp_pallas(xaggp, feat, sW0, sb0, sW1, sb1, mW0, mb0, mW1, mb1, mW2, mb2, mW3, mb3):
    row = lambda i: (i, 0)
    rep2 = lambda i: (0, 0)
    rep3 = lambda i: (0, i, 0)
    return pl.pallas_call(
        _mlp_body,
        grid=(N // BLK_N,),
        in_specs=[
            pl.BlockSpec((2, BLK_N, D), rep3),
            pl.BlockSpec((BLK_N, D), row),
            pl.BlockSpec((D, D), rep2), pl.BlockSpec((1, D), rep2),
            pl.BlockSpec((D, D), rep2), pl.BlockSpec((1, D), rep2),
            pl.BlockSpec((2 * D, D), rep2), pl.BlockSpec((1, D), rep2),
            pl.BlockSpec((D, D), rep2), pl.BlockSpec((1, D), rep2),
            pl.BlockSpec((D, D), rep2), pl.BlockSpec((1, D), rep2),
            pl.BlockSpec((D, D), rep2), pl.BlockSpec((1, D), rep2),
        ],
        out_specs=pl.BlockSpec((BLK_N, D), row),
        out_shape=jax.ShapeDtypeStruct((N, D), jnp.float32),
    )(xaggp, feat, sW0, sb0.reshape(1, D), sW1, sb1.reshape(1, D),
      mW0, mb0.reshape(1, D), mW1, mb1.reshape(1, D), mW2, mb2.reshape(1, D),
      mW3, mb3.reshape(1, D))


def kernel(feat, edge_index, offsets, kW0, kb0, kW1, kb1, sW0, sb0, sW1, sb1,
           mW0, mb0, mW1, mb1, mW2, mb2, mW3, mb3):
    src = edge_index[0]
    dst = edge_index[1]
    ox = offsets[:, 0]
    oy = offsets[:, 1]
    src3d = src.reshape(NC * NS, NCHUNK_B, CB, EB)
    dst3d = dst.reshape(NC * NS, NCHUNK_B, CB, EB)

    sums = _sc_scalar(dst, ox, oy)
    kw, t = _kw_pallas(offsets, sums, kW0, kb0, kW1, kb1)
    w = _sc_weights(t.reshape(NP), dst, ox, oy)
    xaggp = _sc_vector(w, src3d, dst3d, feat, kw)

    return _mlp_pallas(xaggp, feat, sW0, sb0, sW1, sb1,
                       mW0, mb0, mW1, mb1, mW2, mb2, mW3, mb3)
